# baseline (speedup 1.0000x reference)
import functools

import jax
import jax.numpy as jnp
from jax import lax
from jax.experimental import pallas as pl
from jax.experimental.pallas import tpu as pltpu
from jax.experimental.pallas import tpu_sc as plsc

NC, NS, L = 2, 16, 16
NW = NC * NS
B = 16384
H = 64
BPW = B // NW
NCH = 4
CH = BPW // NCH
BLKS_PER_CH = CH // L
NBLK = BPW // L

_MESH = plsc.VectorSubcoreMesh(core_axis_name="c", subcore_axis_name="s")


def _mf_body(users, items, user_emb, item_emb, user_bias, item_bias, out,
             idx_u, idx_v, rows_u, rows_v, bu, bv, out_v, sem):
    wid = lax.axis_index("s") * NC + lax.axis_index("c")

    pltpu.sync_copy(users.at[wid], idx_u)
    pltpu.sync_copy(items.at[wid], idx_v)

    copies = []
    for k in range(NCH):
        copies.append(pltpu.async_copy(user_emb.at[idx_u.at[k]], rows_u.at[k], sem))
        copies.append(pltpu.async_copy(item_emb.at[idx_v.at[k]], rows_v.at[k], sem))
        copies.append(pltpu.async_copy(user_bias.at[idx_u.at[k]], bu.at[k], sem))
        copies.append(pltpu.async_copy(item_bias.at[idx_v.at[k]], bv.at[k], sem))
    for c in copies:
        c.wait()

    def blk(b, carry):
        iota = lax.iota(jnp.int32, L)
        one_hot = [(iota == i).astype(jnp.float32) for i in range(L)]
        k = b // BLKS_PER_CH
        rb = (b % BLKS_PER_CH) * L
        acc = bu[k, pl.ds(rb, L)] + bv[k, pl.ds(rb, L)]
        for i in range(L):
            r = rb + i
            s = rows_u[k, r, pl.ds(0, L)] * rows_v[k, r, pl.ds(0, L)]
            for j in range(1, H // L):
                s = s + rows_u[k, r, pl.ds(j * L, L)] * rows_v[k, r, pl.ds(j * L, L)]
            acc = acc + jnp.sum(s) * one_hot[i]
        out_v[pl.ds(b * L, L)] = acc
        return carry

    lax.fori_loop(0, NBLK, blk, 0)
    pltpu.sync_copy(out_v, out.at[wid])


_mf_sc = functools.partial(
    pl.kernel,
    out_type=jax.ShapeDtypeStruct((NW, BPW), jnp.float32),
    mesh=_MESH,
    compiler_params=pltpu.CompilerParams(
        use_tc_tiling_on_sc=False, needs_layout_passes=False),
    scratch_types=[
        pltpu.VMEM((NCH, CH), jnp.int32),
        pltpu.VMEM((NCH, CH), jnp.int32),
        pltpu.VMEM((NCH, CH, H), jnp.float32),
        pltpu.VMEM((NCH, CH, H), jnp.float32),
        pltpu.VMEM((NCH, CH), jnp.float32),
        pltpu.VMEM((NCH, CH), jnp.float32),
        pltpu.VMEM((BPW,), jnp.float32),
        pltpu.SemaphoreType.DMA,
    ],
)(_mf_body)


def kernel(users, items, user_emb, item_emb, user_bias, item_bias):
    users2 = users.reshape(NW, NCH, CH)
    items2 = items.reshape(NW, NCH, CH)
    out = _mf_sc(users2, items2, user_emb, item_emb, user_bias, item_bias)
    return out.reshape(B)
